# Initial kernel scaffold; baseline (speedup 1.0000x reference)
import jax
import jax.numpy as jnp
from jax.experimental import pallas as pl


def kernel(x, motifs_all, motifs_num, w_att0, b_att0, W0, b0, w_att1, b_att1, W1, b1):
    raise NotImplementedError("write your pallas kernel here")


# speedup vs baseline: 1.3485x; 1.3485x over previous
import functools

import jax
import jax.numpy as jnp
from jax.experimental import pallas as pl
from jax.experimental.pallas import tpu as pltpu


def _layer_kernel(a0_ref, a1_ref, x_ref, nrm_ref, watt_ref, batt_ref,
                  w_ref, b_ref, o_ref):
    x = x_ref[...]
    t0 = jnp.dot(a0_ref[0].astype(jnp.bfloat16), x.astype(jnp.bfloat16),
                 preferred_element_type=jnp.float32)
    t1 = jnp.dot(a1_ref[0].astype(jnp.bfloat16), x.astype(jnp.bfloat16),
                 preferred_element_type=jnp.float32)
    nrm = nrm_ref[...]
    t0 = t0 / nrm[:, 0:1]
    t1 = t1 / nrm[:, 1:2]
    watt = watt_ref[...]
    batt = batt_ref[0, 0]
    l0 = jnp.dot(t0, watt, preferred_element_type=jnp.float32) + batt
    l1 = jnp.dot(t1, watt, preferred_element_type=jnp.float32) + batt
    mx = jnp.maximum(l0, l1)
    e0 = jnp.exp(l0 - mx)
    e1 = jnp.exp(l1 - mx)
    comb = (t0 * e0 + t1 * e1) / (e0 + e1)
    out = jnp.dot(comb, w_ref[...], preferred_element_type=jnp.float32)
    o_ref[...] = jnp.maximum(out + b_ref[...], 0.0)


def _layer(x, motifs_all, nrm_t, w_att, b_att, w, b, *, block_rows,
           interpret=False):
    n = x.shape[0]
    d_in = x.shape[1]
    d_out = w.shape[1]
    m = nrm_t.shape[1]
    grid = (n // block_rows,)
    return pl.pallas_call(
        _layer_kernel,
        grid=grid,
        in_specs=[
            pl.BlockSpec((1, block_rows, n), lambda i: (0, i, 0)),
            pl.BlockSpec((1, block_rows, n), lambda i: (1, i, 0)),
            pl.BlockSpec((n, d_in), lambda i: (0, 0)),
            pl.BlockSpec((block_rows, m), lambda i: (i, 0)),
            pl.BlockSpec((d_in, 1), lambda i: (0, 0)),
            pl.BlockSpec((1, 1), lambda i: (0, 0)),
            pl.BlockSpec((d_in, d_out), lambda i: (0, 0)),
            pl.BlockSpec((1, d_out), lambda i: (0, 0)),
        ],
        out_specs=pl.BlockSpec((block_rows, d_out), lambda i: (i, 0)),
        out_shape=jax.ShapeDtypeStruct((n, d_out), jnp.float32),
        compiler_params=pltpu.CompilerParams(
            dimension_semantics=("arbitrary",)),
        interpret=interpret,
    )(motifs_all, motifs_all, x, nrm_t, w_att, b_att, w, b)


@jax.jit
def kernel(x, motifs_all, motifs_num, w_att0, b_att0, W0, b0,
           w_att1, b_att1, W1, b1):
    nrm_t = motifs_num.T
    b_att0 = b_att0.reshape(1, 1)
    b_att1 = b_att1.reshape(1, 1)
    b0 = b0.reshape(1, -1)
    b1 = b1.reshape(1, -1)
    h = _layer(x, motifs_all, nrm_t, w_att0, b_att0, W0, b0, block_rows=256)
    return _layer(h, motifs_all, nrm_t, w_att1, b_att1, W1, b1,
                  block_rows=256)
